# Initial kernel scaffold; baseline (speedup 1.0000x reference)
import jax
import jax.numpy as jnp
from jax.experimental import pallas as pl


def kernel(x, edge_index, edge_attr, W1, W2, W3, W4, Wc, bias):
    raise NotImplementedError("write your pallas kernel here")


# speedup vs baseline: 2.3427x; 2.3427x over previous
import functools

import jax
import jax.numpy as jnp
from jax import lax
from jax.experimental import pallas as pl
from jax.experimental.pallas import tpu as pltpu
from jax.experimental.pallas import tpu_sc as plsc

N_NODES = 10000
N_EDGES = 320000
D_IN = 128
D_OUT = 128
K = 4

_E_BLK = 8000


def _edge_mlp_body(attr_ref, w1_ref, w2_ref, w3_ref, w4a_ref, w4b_ref, out_ref):
    a = attr_ref[...]
    lin = jnp.maximum(jnp.dot(a, w1_ref[...], preferred_element_type=jnp.float32), 0.0)
    g = jnp.tanh(jnp.dot(a, w2_ref[...], preferred_element_type=jnp.float32)) * jnp.tanh(
        jnp.dot(a, w3_ref[...], preferred_element_type=jnp.float32)
    )
    acc = jnp.dot(lin, w4a_ref[...], preferred_element_type=jnp.float32) + jnp.dot(
        g, w4b_ref[...], preferred_element_type=jnp.float32
    )
    out_ref[...] = jnp.maximum(acc, 0.0)


def _edge_mlp(edge_attr, w1t, w2t, w3t, w4a, w4b):
    n_in = edge_attr.shape[1]
    grid = N_EDGES // _E_BLK
    return pl.pallas_call(
        _edge_mlp_body,
        grid=(grid,),
        in_specs=[
            pl.BlockSpec((_E_BLK, n_in), lambda i: (i, 0)),
            pl.BlockSpec(w1t.shape, lambda i: (0, 0)),
            pl.BlockSpec(w2t.shape, lambda i: (0, 0)),
            pl.BlockSpec(w3t.shape, lambda i: (0, 0)),
            pl.BlockSpec(w4a.shape, lambda i: (0, 0)),
            pl.BlockSpec(w4b.shape, lambda i: (0, 0)),
        ],
        out_specs=pl.BlockSpec((_E_BLK, K * _L), lambda i: (i, 0)),
        out_shape=jax.ShapeDtypeStruct((N_EDGES, K * _L), jnp.float32),
    )(edge_attr, w1t, w2t, w3t, w4a, w4b)


_N_BLK = 2000


def _zmm_body(x_ref, w_ref, out_ref):
    out_ref[...] = jnp.dot(x_ref[...], w_ref[...], preferred_element_type=jnp.float32)


def _z_matmul(x, wflat):
    grid = N_NODES // _N_BLK
    return pl.pallas_call(
        _zmm_body,
        grid=(grid,),
        in_specs=[
            pl.BlockSpec((_N_BLK, D_IN), lambda i: (i, 0)),
            pl.BlockSpec((D_IN, K * D_OUT), lambda i: (0, 0)),
        ],
        out_specs=pl.BlockSpec((_N_BLK, K * D_OUT), lambda i: (i, 0)),
        out_shape=jax.ShapeDtypeStruct((N_NODES, K * D_OUT), jnp.float32),
    )(x, wflat)


_NC = 2
_NS = 16
_NW = _NC * _NS
_L = 16
_EPW = N_EDGES // _NW
_CH = 40
_NCHUNK = _EPW // _CH
_ROWS_PT = 640
_N_PAD = _NS * _ROWS_PT


def _sc_body(z_hbm, src_hbm, dst_hbm, ea_hbm, zero_hbm, out_hbm,
             src_v, dst_v, ea_v, zrow_v, msg_v, acc_sh, sem):
    c = lax.axis_index("c")
    s = lax.axis_index("s")
    wid = s * _NC + c

    pltpu.sync_copy(zero_hbm.at[pl.ds(s * _ROWS_PT, _ROWS_PT)],
                    acc_sh.at[pl.ds(s * _ROWS_PT, _ROWS_PT)])
    plsc.subcore_barrier()

    base0 = wid * _EPW

    def chunk_body(k, carry):
        base = base0 + k * _CH
        pltpu.sync_copy(src_hbm.at[pl.ds(base, _CH)], src_v)
        pltpu.sync_copy(dst_hbm.at[pl.ds(base, _CH)], dst_v)
        pltpu.sync_copy(ea_hbm.at[pl.ds(base, _CH)], ea_v)
        pltpu.async_copy(z_hbm.at[src_v], zrow_v, sem).wait()

        def edge_body(e, carry2):
            w0 = ea_v[e, pl.ds(0, _L)]
            w1 = ea_v[e, pl.ds(_L, _L)]
            w2 = ea_v[e, pl.ds(2 * _L, _L)]
            w3 = ea_v[e, pl.ds(3 * _L, _L)]
            for j in range(D_OUT // _L):
                off = j * _L
                v = (w0 * zrow_v[e, pl.ds(off, _L)]
                     + w1 * zrow_v[e, pl.ds(D_OUT + off, _L)]
                     + w2 * zrow_v[e, pl.ds(2 * D_OUT + off, _L)]
                     + w3 * zrow_v[e, pl.ds(3 * D_OUT + off, _L)])
                msg_v[e, pl.ds(off, _L)] = v
            return carry2

        lax.fori_loop(0, _CH, edge_body, 0, unroll=2)
        pltpu.sync_copy(msg_v, acc_sh.at[dst_v], add=True)
        return carry

    lax.fori_loop(0, _NCHUNK, chunk_body, 0)
    plsc.subcore_barrier()
    pltpu.sync_copy(acc_sh.at[pl.ds(s * _ROWS_PT, _ROWS_PT)],
                    out_hbm.at[c, pl.ds(s * _ROWS_PT, _ROWS_PT)])


def _sc_scatter(z, src, dst, ea, zeros):
    mesh = plsc.VectorSubcoreMesh(core_axis_name="c", subcore_axis_name="s")
    fn = functools.partial(
        pl.kernel,
        mesh=mesh,
        out_type=jax.ShapeDtypeStruct((_NC, _N_PAD, D_OUT), jnp.float32),
        scratch_types=[
            pltpu.VMEM((_CH,), jnp.int32),
            pltpu.VMEM((_CH,), jnp.int32),
            pltpu.VMEM((_CH, K * _L), jnp.float32),
            pltpu.VMEM((_CH, K * D_OUT), jnp.float32),
            pltpu.VMEM((_CH, D_OUT), jnp.float32),
            pltpu.VMEM_SHARED((_N_PAD, D_OUT), jnp.float32),
            pltpu.SemaphoreType.DMA,
        ],
    )(_sc_body)
    return fn(z, src, dst, ea, zeros)


def _final_body(p_ref, b_ref, out_ref):
    out_ref[...] = jnp.maximum(p_ref[0] + p_ref[1] + b_ref[...], 0.0)


def _final(partials, bias2d):
    grid = N_NODES // _N_BLK
    return pl.pallas_call(
        _final_body,
        grid=(grid,),
        in_specs=[
            pl.BlockSpec((_NC, _N_BLK, D_OUT), lambda i: (0, i, 0)),
            pl.BlockSpec((1, D_OUT), lambda i: (0, 0)),
        ],
        out_specs=pl.BlockSpec((_N_BLK, D_OUT), lambda i: (i, 0)),
        out_shape=jax.ShapeDtypeStruct((N_NODES, D_OUT), jnp.float32),
    )(partials, bias2d)


def kernel(x, edge_index, edge_attr, W1, W2, W3, W4, Wc, bias):
    src = edge_index[0].astype(jnp.int32)
    dst = edge_index[1].astype(jnp.int32)
    w1t = W1.T
    w2t = W2.T
    w3t = W3.T
    w4t = jnp.repeat(W4.T, _L, axis=1)
    w4a = w4t[: 2 * 16]
    w4b = w4t[2 * 16:]
    wflat = jnp.transpose(Wc, (1, 0, 2)).reshape(D_IN, K * D_OUT)

    ea = _edge_mlp(edge_attr, w1t, w2t, w3t, w4a, w4b)
    z = _z_matmul(x, wflat)
    zeros = jnp.zeros((_N_PAD, D_OUT), jnp.float32)
    partials = _sc_scatter(z, src, dst, ea, zeros)
    return _final(partials, bias.reshape(1, D_OUT))


# speedup vs baseline: 3.2431x; 1.3843x over previous
import functools

import jax
import jax.numpy as jnp
from jax import lax
from jax.experimental import pallas as pl
from jax.experimental.pallas import tpu as pltpu
from jax.experimental.pallas import tpu_sc as plsc

N_NODES = 10000
N_EDGES = 320000
D_IN = 128
D_OUT = 128
K = 4

_E_BLK = 8000


def _edge_mlp_body(attr_ref, w1_ref, w2_ref, w3_ref, w4a_ref, w4b_ref, out_ref):
    a = attr_ref[...]
    lin = jnp.maximum(jnp.dot(a, w1_ref[...], preferred_element_type=jnp.float32), 0.0)
    g = jnp.tanh(jnp.dot(a, w2_ref[...], preferred_element_type=jnp.float32)) * jnp.tanh(
        jnp.dot(a, w3_ref[...], preferred_element_type=jnp.float32)
    )
    acc = jnp.dot(lin, w4a_ref[...], preferred_element_type=jnp.float32) + jnp.dot(
        g, w4b_ref[...], preferred_element_type=jnp.float32
    )
    out_ref[...] = jnp.maximum(acc, 0.0)


def _edge_mlp(edge_attr, w1t, w2t, w3t, w4a, w4b):
    n_in = edge_attr.shape[1]
    grid = N_EDGES // _E_BLK
    return pl.pallas_call(
        _edge_mlp_body,
        grid=(grid,),
        in_specs=[
            pl.BlockSpec((_E_BLK, n_in), lambda i: (i, 0)),
            pl.BlockSpec(w1t.shape, lambda i: (0, 0)),
            pl.BlockSpec(w2t.shape, lambda i: (0, 0)),
            pl.BlockSpec(w3t.shape, lambda i: (0, 0)),
            pl.BlockSpec(w4a.shape, lambda i: (0, 0)),
            pl.BlockSpec(w4b.shape, lambda i: (0, 0)),
        ],
        out_specs=pl.BlockSpec((_E_BLK, K * _L), lambda i: (i, 0)),
        out_shape=jax.ShapeDtypeStruct((N_EDGES, K * _L), jnp.float32),
    )(edge_attr, w1t, w2t, w3t, w4a, w4b)


_N_BLK = 2000


def _zmm_body(x_ref, w_ref, out_ref):
    out_ref[...] = jnp.dot(
        x_ref[...], w_ref[...], preferred_element_type=jnp.float32
    ).astype(jnp.bfloat16)


def _z_matmul(x, wflat):
    grid = N_NODES // _N_BLK
    return pl.pallas_call(
        _zmm_body,
        grid=(grid,),
        in_specs=[
            pl.BlockSpec((_N_BLK, D_IN), lambda i: (i, 0)),
            pl.BlockSpec((D_IN, K * D_OUT), lambda i: (0, 0)),
        ],
        out_specs=pl.BlockSpec((_N_BLK, K * D_OUT), lambda i: (i, 0)),
        out_shape=jax.ShapeDtypeStruct((N_NODES, K * D_OUT), jnp.bfloat16),
    )(x, wflat)


_NC = 2
_NS = 16
_NW = _NC * _NS
_L = 16
_CH = 32
_E_PAD = 327680
_NCH_G = _E_PAD // _CH
_NCHT = _NCH_G // _NW
_ROWS_PT = 632
_N_PAD = _NS * _ROWS_PT


def _sc_body(z_hbm, src_hbm, dst_hbm, wts_hbm, zero_hbm, out_hbm,
             src0, src1, src2, src3, dst0, dst1, dst2, dst3,
             wts0, wts1, zrow0, zrow1, msg0, msg1, acc_sh,
             rsem0, rsem1, rsem2, rsem3, dsem0, dsem1, dsem2, dsem3,
             wsem0, wsem1, gsem0, gsem1, ssem0, ssem1):
    c = lax.axis_index("c")
    s = lax.axis_index("s")
    wid = s * _NC + c
    srcs = (src0, src1, src2, src3)
    rsems = (rsem0, rsem1, rsem2, rsem3)
    dsts = (dst0, dst1, dst2, dst3)
    dsems = (dsem0, dsem1, dsem2, dsem3)
    wtss = (wts0, wts1)
    wsems = (wsem0, wsem1)
    zrows = (zrow0, zrow1)
    gsems = (gsem0, gsem1)
    msgs = (msg0, msg1)
    ssems = (ssem0, ssem1)

    pltpu.sync_copy(zero_hbm.at[pl.ds(s * _ROWS_PT, _ROWS_PT)],
                    acc_sh.at[pl.ds(s * _ROWS_PT, _ROWS_PT)])
    plsc.subcore_barrier()

    base = wid * _NCHT
    ebase = base * _CH
    pltpu.async_copy(src_hbm.at[pl.ds(ebase, _CH)], srcs[0], rsems[0])
    pltpu.async_copy(src_hbm.at[pl.ds(ebase + _CH, _CH)], srcs[1], rsems[1])
    pltpu.async_copy(dst_hbm.at[pl.ds(ebase, _CH)], dsts[0], dsems[0])
    pltpu.async_copy(dst_hbm.at[pl.ds(ebase + _CH, _CH)], dsts[1], dsems[1])
    pltpu.async_copy(wts_hbm.at[base], wtss[0], wsems[0])
    pltpu.make_async_copy(src_hbm.at[pl.ds(ebase, _CH)], srcs[0],
                          rsems[0]).wait()
    pltpu.async_copy(z_hbm.at[srcs[0]], zrows[0], gsems[0])

    def super_body(k2, carry):
        k0 = k2 * 4
        for j in range(4):
            k = k0 + j
            b2 = j % 2
            b4 = j % 4
            nx2 = (j + 1) % 2
            nx4 = (j + 1) % 4
            pf4 = (j + 2) % 4
            pv4 = (j + 3) % 4

            @pl.when(k >= 2)
            def _():
                pltpu.make_async_copy(msgs[b2], acc_sh.at[dsts[pf4]],
                                      ssems[b2]).wait()

            @pl.when(k + 2 < _NCHT)
            def _():
                off = ebase + (k + 2) * _CH
                pltpu.async_copy(src_hbm.at[pl.ds(off, _CH)], srcs[pf4],
                                 rsems[pf4])
                pltpu.async_copy(dst_hbm.at[pl.ds(off, _CH)], dsts[pf4],
                                 dsems[pf4])

            @pl.when(k + 1 < _NCHT)
            def _():
                pltpu.make_async_copy(
                    src_hbm.at[pl.ds(ebase + (k + 1) * _CH, _CH)], srcs[nx4],
                    rsems[nx4]).wait()
                pltpu.async_copy(z_hbm.at[srcs[nx4]], zrows[nx2], gsems[nx2])
                pltpu.async_copy(wts_hbm.at[base + k + 1], wtss[nx2],
                                 wsems[nx2])

            pltpu.make_async_copy(z_hbm.at[srcs[b4]], zrows[b2],
                                  gsems[b2]).wait()
            pltpu.make_async_copy(wts_hbm.at[base + k], wtss[b2],
                                  wsems[b2]).wait()
            pltpu.make_async_copy(dst_hbm.at[pl.ds(ebase + k * _CH, _CH)],
                                  dsts[b4], dsems[b4]).wait()

            zr = zrows[b2]
            mg = msgs[b2]
            wt = wtss[b2]

            def edge_body(e, carry2):
                rw = e // 2
                cw = (e % 2) * (K * _L)
                w0 = wt[rw, pl.ds(cw, _L)]
                w1 = wt[rw, pl.ds(cw + _L, _L)]
                w2 = wt[rw, pl.ds(cw + 2 * _L, _L)]
                w3 = wt[rw, pl.ds(cw + 3 * _L, _L)]
                for t in range(D_OUT // (2 * _L)):
                    def halves(i):
                        pr = plsc.bitcast(
                            zr[e, pl.ds(i * (D_OUT // 2) + _L * t, _L)],
                            jnp.bfloat16)
                        return plsc.unpack(
                            pr, format=plsc.PackFormat.INTERLEAVED)

                    a0, b0 = halves(0)
                    a1, b1 = halves(1)
                    a2, b2 = halves(2)
                    a3, b3 = halves(3)
                    va = w0 * a0 + w1 * a1 + w2 * a2 + w3 * a3
                    vb = w0 * b0 + w1 * b1 + w2 * b2 + w3 * b3
                    mg[e, pl.ds(2 * _L * t, _L)] = va
                    mg[e, pl.ds(2 * _L * t + _L, _L)] = vb
                return carry2

            lax.fori_loop(0, _CH, edge_body, 0, unroll=2)

            pltpu.async_copy(msgs[b2], acc_sh.at[dsts[b4]],
                             ssems[b2], add=True)
        return carry

    lax.fori_loop(0, _NCHT // 4, super_body, 0)
    pltpu.make_async_copy(msgs[0], acc_sh.at[dsts[(_NCHT - 2) % 4]],
                          ssems[0]).wait()
    pltpu.make_async_copy(msgs[1], acc_sh.at[dsts[(_NCHT - 1) % 4]],
                          ssems[1]).wait()
    plsc.subcore_barrier()
    pltpu.sync_copy(acc_sh.at[pl.ds(s * _ROWS_PT, _ROWS_PT)],
                    out_hbm.at[c, pl.ds(s * _ROWS_PT, _ROWS_PT)])


def _sc_scatter(z, srcp, dstp, wts, zeros):
    mesh = plsc.VectorSubcoreMesh(core_axis_name="c", subcore_axis_name="s")
    fn = functools.partial(
        pl.kernel,
        mesh=mesh,
        compiler_params=pltpu.CompilerParams(needs_layout_passes=False),
        out_type=jax.ShapeDtypeStruct((_NC, _N_PAD, D_OUT), jnp.float32),
        scratch_types=(
            [pltpu.VMEM((_CH,), jnp.int32)] * 8
            + [pltpu.VMEM((_CH // 2, 2 * K * _L), jnp.float32)] * 2
            + [pltpu.VMEM((_CH, K * D_OUT // 2), jnp.int32)] * 2
            + [pltpu.VMEM((_CH, D_OUT), jnp.float32)] * 2
            + [pltpu.VMEM_SHARED((_N_PAD, D_OUT), jnp.float32)]
            + [pltpu.SemaphoreType.DMA] * 14
        ),
    )(_sc_body)
    return fn(z, srcp, dstp, wts, zeros)


def _final_body(p_ref, b_ref, out_ref):
    out_ref[...] = jnp.maximum(p_ref[0] + p_ref[1] + b_ref[...], 0.0)


def _final(partials, bias2d):
    grid = N_NODES // _N_BLK
    return pl.pallas_call(
        _final_body,
        grid=(grid,),
        in_specs=[
            pl.BlockSpec((_NC, _N_BLK, D_OUT), lambda i: (0, i, 0)),
            pl.BlockSpec((1, D_OUT), lambda i: (0, 0)),
        ],
        out_specs=pl.BlockSpec((_N_BLK, D_OUT), lambda i: (i, 0)),
        out_shape=jax.ShapeDtypeStruct((N_NODES, D_OUT), jnp.float32),
    )(partials, bias2d)


def kernel(x, edge_index, edge_attr, W1, W2, W3, W4, Wc, bias):
    src = edge_index[0].astype(jnp.int32)
    dst = edge_index[1].astype(jnp.int32)
    w1t = W1.T
    w2t = W2.T
    w3t = W3.T
    w4t = jnp.repeat(W4.T, _L, axis=1)
    w4a = w4t[: 2 * 16]
    w4b = w4t[2 * 16:]
    wflat = jnp.transpose(Wc, (1, 0, 2)).reshape(D_IN, K * D_OUT)
    pos = jnp.arange(K * D_OUT)
    blk, r = pos // (2 * _L), pos % (2 * _L)
    wflat = wflat[:, blk * (2 * _L) + (r % 2) * _L + r // 2]

    ea = _edge_mlp(edge_attr, w1t, w2t, w3t, w4a, w4b)
    zb = _z_matmul(x, wflat)
    z = jax.lax.bitcast_convert_type(
        zb.reshape(N_NODES, K * D_OUT // 2, 2), jnp.int32
    )

    npad = _E_PAD - N_EDGES
    srcp = jnp.concatenate([src, jnp.zeros((npad,), jnp.int32)])
    dstp = jnp.concatenate([dst, jnp.zeros((npad,), jnp.int32)])
    eap = jnp.concatenate([ea, jnp.zeros((npad, K * _L), jnp.float32)])
    wts = eap.reshape(_NCH_G, _CH // 2, 2 * K * _L)

    zeros = jnp.zeros((_N_PAD, D_OUT), jnp.float32)
    partials = _sc_scatter(z, srcp, dstp, wts, zeros)
    return _final(partials, bias.reshape(1, D_OUT))


# speedup vs baseline: 4.0296x; 1.2425x over previous
import functools

import jax
import jax.numpy as jnp
from jax import lax
from jax.experimental import pallas as pl
from jax.experimental.pallas import tpu as pltpu
from jax.experimental.pallas import tpu_sc as plsc

N_NODES = 10000
N_EDGES = 320000
D_IN = 128
D_OUT = 128
K = 4

_E_BLK = 8000


def _edge_mlp_body(attr_ref, w1_ref, w2_ref, w3_ref, w4a_ref, w4b_ref, out_ref):
    a = attr_ref[...]
    lin = jnp.maximum(jnp.dot(a, w1_ref[...], preferred_element_type=jnp.float32), 0.0)
    g = jnp.tanh(jnp.dot(a, w2_ref[...], preferred_element_type=jnp.float32)) * jnp.tanh(
        jnp.dot(a, w3_ref[...], preferred_element_type=jnp.float32)
    )
    acc = jnp.dot(lin, w4a_ref[...], preferred_element_type=jnp.float32) + jnp.dot(
        g, w4b_ref[...], preferred_element_type=jnp.float32
    )
    out_ref[...] = jnp.maximum(acc, 0.0)


def _edge_mlp(edge_attr, w1t, w2t, w3t, w4a, w4b):
    n_in = edge_attr.shape[1]
    grid = N_EDGES // _E_BLK
    return pl.pallas_call(
        _edge_mlp_body,
        grid=(grid,),
        in_specs=[
            pl.BlockSpec((_E_BLK, n_in), lambda i: (i, 0)),
            pl.BlockSpec(w1t.shape, lambda i: (0, 0)),
            pl.BlockSpec(w2t.shape, lambda i: (0, 0)),
            pl.BlockSpec(w3t.shape, lambda i: (0, 0)),
            pl.BlockSpec(w4a.shape, lambda i: (0, 0)),
            pl.BlockSpec(w4b.shape, lambda i: (0, 0)),
        ],
        out_specs=pl.BlockSpec((_E_BLK, K * _L), lambda i: (i, 0)),
        out_shape=jax.ShapeDtypeStruct((N_EDGES, K * _L), jnp.float32),
    )(edge_attr, w1t, w2t, w3t, w4a, w4b)


_N_BLK = 2000


def _zmm_body(x_ref, w_ref, out_ref):
    out_ref[...] = jnp.dot(
        x_ref[...], w_ref[...], preferred_element_type=jnp.float32
    ).astype(jnp.bfloat16)


def _z_matmul(x, wflat):
    grid = N_NODES // _N_BLK
    return pl.pallas_call(
        _zmm_body,
        grid=(grid,),
        in_specs=[
            pl.BlockSpec((_N_BLK, D_IN), lambda i: (i, 0)),
            pl.BlockSpec((D_IN, K * D_OUT), lambda i: (0, 0)),
        ],
        out_specs=pl.BlockSpec((_N_BLK, K * D_OUT), lambda i: (i, 0)),
        out_shape=jax.ShapeDtypeStruct((N_NODES, K * D_OUT), jnp.bfloat16),
    )(x, wflat)


_NC = 2
_NS = 16
_NW = _NC * _NS
_L = 16
_CH = 40
_NCH_G = N_EDGES // _CH
_NCHT = _NCH_G // _NW
_ROWS_PT = 632
_N_PAD = _NS * _ROWS_PT


def _sc_body(z_hbm, src_hbm, dst_hbm, wts_hbm, zero_hbm, out_hbm,
             src0, src1, src2, src3, dst0, dst1, dst2, dst3,
             wts0, wts1, zrow0, zrow1, msg0, msg1, acc_sh,
             rsem0, rsem1, rsem2, rsem3, dsem0, dsem1, dsem2, dsem3,
             wsem0, wsem1, gsem0, gsem1, ssem0, ssem1):
    c = lax.axis_index("c")
    s = lax.axis_index("s")
    wid = s * _NC + c
    srcs = (src0, src1, src2, src3)
    rsems = (rsem0, rsem1, rsem2, rsem3)
    dsts = (dst0, dst1, dst2, dst3)
    dsems = (dsem0, dsem1, dsem2, dsem3)
    wtss = (wts0, wts1)
    wsems = (wsem0, wsem1)
    zrows = (zrow0, zrow1)
    gsems = (gsem0, gsem1)
    msgs = (msg0, msg1)
    ssems = (ssem0, ssem1)

    pltpu.sync_copy(zero_hbm.at[pl.ds(s * _ROWS_PT, _ROWS_PT)],
                    acc_sh.at[pl.ds(s * _ROWS_PT, _ROWS_PT)])
    plsc.subcore_barrier()

    base = wid * _NCHT
    ebase = base * _CH
    pltpu.async_copy(src_hbm.at[pl.ds(ebase, _CH)], srcs[0], rsems[0])
    pltpu.async_copy(src_hbm.at[pl.ds(ebase + _CH, _CH)], srcs[1], rsems[1])
    pltpu.async_copy(dst_hbm.at[pl.ds(ebase, _CH)], dsts[0], dsems[0])
    pltpu.async_copy(dst_hbm.at[pl.ds(ebase + _CH, _CH)], dsts[1], dsems[1])
    pltpu.async_copy(wts_hbm.at[base], wtss[0], wsems[0])
    pltpu.make_async_copy(src_hbm.at[pl.ds(ebase, _CH)], srcs[0],
                          rsems[0]).wait()
    pltpu.async_copy(z_hbm.at[srcs[0]], zrows[0], gsems[0])

    def chunk_step(k, j):
        if True:
            b2 = j % 2
            b4 = j % 4
            nx2 = (j + 1) % 2
            nx4 = (j + 1) % 4
            pf4 = (j + 2) % 4

            @pl.when(k >= 2)
            def _():
                pltpu.make_async_copy(msgs[b2], acc_sh.at[dsts[pf4]],
                                      ssems[b2]).wait()

            @pl.when(k + 2 < _NCHT)
            def _():
                off = ebase + (k + 2) * _CH
                pltpu.async_copy(src_hbm.at[pl.ds(off, _CH)], srcs[pf4],
                                 rsems[pf4])
                pltpu.async_copy(dst_hbm.at[pl.ds(off, _CH)], dsts[pf4],
                                 dsems[pf4])

            @pl.when(k + 1 < _NCHT)
            def _():
                pltpu.make_async_copy(
                    src_hbm.at[pl.ds(ebase + (k + 1) * _CH, _CH)], srcs[nx4],
                    rsems[nx4]).wait()
                pltpu.async_copy(z_hbm.at[srcs[nx4]], zrows[nx2], gsems[nx2])
                pltpu.async_copy(wts_hbm.at[base + k + 1], wtss[nx2],
                                 wsems[nx2])

            pltpu.make_async_copy(z_hbm.at[srcs[b4]], zrows[b2],
                                  gsems[b2]).wait()
            pltpu.make_async_copy(wts_hbm.at[base + k], wtss[b2],
                                  wsems[b2]).wait()
            pltpu.make_async_copy(dst_hbm.at[pl.ds(ebase + k * _CH, _CH)],
                                  dsts[b4], dsems[b4]).wait()

            zr = zrows[b2]
            mg = msgs[b2]
            wt = wtss[b2]

            def edge_body(e, carry2):
                rw = e // 2
                cw = (e % 2) * (K * _L)
                w0 = wt[rw, pl.ds(cw, _L)]
                w1 = wt[rw, pl.ds(cw + _L, _L)]
                w2 = wt[rw, pl.ds(cw + 2 * _L, _L)]
                w3 = wt[rw, pl.ds(cw + 3 * _L, _L)]
                for t in range(D_OUT // (2 * _L)):
                    def halves(i):
                        pr = plsc.bitcast(
                            zr[e, pl.ds(i * (D_OUT // 2) + _L * t, _L)],
                            jnp.bfloat16)
                        return plsc.unpack(
                            pr, format=plsc.PackFormat.INTERLEAVED)

                    a0, b0 = halves(0)
                    a1, b1 = halves(1)
                    a2, b2 = halves(2)
                    a3, b3 = halves(3)
                    va = w0 * a0 + w1 * a1 + w2 * a2 + w3 * a3
                    vb = w0 * b0 + w1 * b1 + w2 * b2 + w3 * b3
                    mg[e, pl.ds(2 * _L * t, _L)] = va
                    mg[e, pl.ds(2 * _L * t + _L, _L)] = vb
                return carry2

            lax.fori_loop(0, _CH, edge_body, 0, unroll=2)

            pltpu.async_copy(msgs[b2], acc_sh.at[dsts[b4]],
                             ssems[b2], add=True)

    def super_body(k2, carry):
        for j in range(4):
            chunk_step(k2 * 4 + j, j)
        return carry

    ntail = _NCHT % 4
    lax.fori_loop(0, _NCHT // 4, super_body, 0)
    for j in range(ntail):
        chunk_step(jnp.int32((_NCHT // 4) * 4 + j), j)
    pltpu.make_async_copy(msgs[_NCHT % 2], acc_sh.at[dsts[(_NCHT - 2) % 4]],
                          ssems[_NCHT % 2]).wait()
    pltpu.make_async_copy(msgs[(_NCHT + 1) % 2],
                          acc_sh.at[dsts[(_NCHT - 1) % 4]],
                          ssems[(_NCHT + 1) % 2]).wait()
    plsc.subcore_barrier()
    pltpu.sync_copy(acc_sh.at[pl.ds(s * _ROWS_PT, _ROWS_PT)],
                    out_hbm.at[c, pl.ds(s * _ROWS_PT, _ROWS_PT)])


def _sc_scatter(z, srcp, dstp, wts, zeros):
    mesh = plsc.VectorSubcoreMesh(core_axis_name="c", subcore_axis_name="s")
    fn = functools.partial(
        pl.kernel,
        mesh=mesh,
        compiler_params=pltpu.CompilerParams(needs_layout_passes=False),
        out_type=jax.ShapeDtypeStruct((_NC, _N_PAD, D_OUT), jnp.float32),
        scratch_types=(
            [pltpu.VMEM((_CH,), jnp.int32)] * 8
            + [pltpu.VMEM((_CH // 2, 2 * K * _L), jnp.float32)] * 2
            + [pltpu.VMEM((_CH, K * D_OUT // 2), jnp.int32)] * 2
            + [pltpu.VMEM((_CH, D_OUT), jnp.float32)] * 2
            + [pltpu.VMEM_SHARED((_N_PAD, D_OUT), jnp.float32)]
            + [pltpu.SemaphoreType.DMA] * 14
        ),
    )(_sc_body)
    return fn(z, srcp, dstp, wts, zeros)


def _final_body(p_ref, b_ref, out_ref):
    out_ref[...] = jnp.maximum(p_ref[0] + p_ref[1] + b_ref[...], 0.0)


def _final(partials, bias2d):
    grid = N_NODES // _N_BLK
    return pl.pallas_call(
        _final_body,
        grid=(grid,),
        in_specs=[
            pl.BlockSpec((_NC, _N_BLK, D_OUT), lambda i: (0, i, 0)),
            pl.BlockSpec((1, D_OUT), lambda i: (0, 0)),
        ],
        out_specs=pl.BlockSpec((_N_BLK, D_OUT), lambda i: (i, 0)),
        out_shape=jax.ShapeDtypeStruct((N_NODES, D_OUT), jnp.float32),
    )(partials, bias2d)


def kernel(x, edge_index, edge_attr, W1, W2, W3, W4, Wc, bias):
    src = edge_index[0].astype(jnp.int32)
    dst = edge_index[1].astype(jnp.int32)
    w1t = W1.T
    w2t = W2.T
    w3t = W3.T
    w4t = jnp.repeat(W4.T, _L, axis=1)
    w4a = w4t[: 2 * 16]
    w4b = w4t[2 * 16:]
    wflat = jnp.transpose(Wc, (1, 0, 2)).reshape(D_IN, K * D_OUT)
    pos = jnp.arange(K * D_OUT)
    blk, r = pos // (2 * _L), pos % (2 * _L)
    wflat = wflat[:, blk * (2 * _L) + (r % 2) * _L + r // 2]

    ea = _edge_mlp(edge_attr, w1t, w2t, w3t, w4a, w4b)
    zb = _z_matmul(x, wflat)
    z = jax.lax.bitcast_convert_type(
        zb.reshape(N_NODES, K * D_OUT // 2, 2), jnp.int32
    )

    wts = ea.reshape(_NCH_G, _CH // 2, 2 * K * _L)

    zeros = jnp.zeros((_N_PAD, D_OUT), jnp.float32)
    partials = _sc_scatter(z, src, dst, wts, zeros)
    return _final(partials, bias.reshape(1, D_OUT))


# speedup vs baseline: 5.3243x; 1.3213x over previous
import functools

import jax
import jax.numpy as jnp
from jax import lax
from jax.experimental import pallas as pl
from jax.experimental.pallas import tpu as pltpu
from jax.experimental.pallas import tpu_sc as plsc

N_NODES = 10000
N_EDGES = 320000
D_IN = 128
D_OUT = 128
K = 4

_E2 = N_EDGES // 2
_E2_BLK = 4000


def _edge_mlp_body(attr_ref, ws1_ref, ws2_ref, out_ref):
    t = jnp.dot(attr_ref[...], ws1_ref[...], preferred_element_type=jnp.float32)
    la = jnp.maximum(t[:, 0:32], 0.0)
    ga = jnp.tanh(t[:, 32:64]) * jnp.tanh(t[:, 64:96])
    lb = jnp.maximum(t[:, 96:128], 0.0)
    gb = jnp.tanh(t[:, 128:160]) * jnp.tanh(t[:, 160:192])
    tmp = jnp.concatenate([la, ga, lb, gb], axis=1)
    out_ref[...] = jnp.maximum(
        jnp.dot(tmp, ws2_ref[...], preferred_element_type=jnp.float32), 0.0
    )


def _edge_mlp(attr2, ws1, ws2):
    grid = _E2 // _E2_BLK
    return pl.pallas_call(
        _edge_mlp_body,
        grid=(grid,),
        in_specs=[
            pl.BlockSpec((_E2_BLK, 32), lambda i: (i, 0)),
            pl.BlockSpec(ws1.shape, lambda i: (0, 0)),
            pl.BlockSpec(ws2.shape, lambda i: (0, 0)),
        ],
        out_specs=pl.BlockSpec((_E2_BLK, 2 * K * _L), lambda i: (i, 0)),
        out_shape=jax.ShapeDtypeStruct((_E2, 2 * K * _L), jnp.float32),
    )(attr2, ws1, ws2)


_N_BLK = 2000


def _zmm_body(x_ref, w_ref, out_ref):
    zf = jnp.dot(x_ref[...], w_ref[...], preferred_element_type=jnp.float32)
    lo = jax.lax.bitcast_convert_type(
        zf[:, : K * D_OUT // 2].astype(jnp.bfloat16), jnp.uint16
    ).astype(jnp.int32)
    hi = jax.lax.bitcast_convert_type(
        zf[:, K * D_OUT // 2:].astype(jnp.bfloat16), jnp.uint16
    ).astype(jnp.int32)
    out_ref[...] = lo | (hi << 16)


def _z_matmul(x, wflat):
    grid = N_NODES // _N_BLK
    return pl.pallas_call(
        _zmm_body,
        grid=(grid,),
        in_specs=[
            pl.BlockSpec((_N_BLK, D_IN), lambda i: (i, 0)),
            pl.BlockSpec((D_IN, K * D_OUT), lambda i: (0, 0)),
        ],
        out_specs=pl.BlockSpec((_N_BLK, K * D_OUT // 2), lambda i: (i, 0)),
        out_shape=jax.ShapeDtypeStruct((N_NODES, K * D_OUT // 2), jnp.int32),
    )(x, wflat)


_NC = 2
_NS = 16
_NW = _NC * _NS
_L = 16
_CH = 40
_NCH_G = N_EDGES // _CH
_NCHT = _NCH_G // _NW
_ROWS_PT = 632
_N_PAD = _NS * _ROWS_PT


def _sc_body(z_hbm, src_hbm, dst_hbm, wts_hbm, zero_hbm, out_hbm,
             src0, src1, src2, src3, dst0, dst1, dst2, dst3,
             wts0, wts1, zrow0, zrow1, msg0, msg1, acc_sh,
             rsem0, rsem1, rsem2, rsem3, dsem0, dsem1, dsem2, dsem3,
             wsem0, wsem1, gsem0, gsem1, ssem0, ssem1):
    c = lax.axis_index("c")
    s = lax.axis_index("s")
    wid = s * _NC + c
    srcs = (src0, src1, src2, src3)
    rsems = (rsem0, rsem1, rsem2, rsem3)
    dsts = (dst0, dst1, dst2, dst3)
    dsems = (dsem0, dsem1, dsem2, dsem3)
    wtss = (wts0, wts1)
    wsems = (wsem0, wsem1)
    zrows = (zrow0, zrow1)
    gsems = (gsem0, gsem1)
    msgs = (msg0, msg1)
    ssems = (ssem0, ssem1)

    pltpu.sync_copy(zero_hbm.at[pl.ds(s * _ROWS_PT, _ROWS_PT)],
                    acc_sh.at[pl.ds(s * _ROWS_PT, _ROWS_PT)])
    plsc.subcore_barrier()

    base = wid * _NCHT
    ebase = base * _CH
    pltpu.async_copy(src_hbm.at[pl.ds(ebase, _CH)], srcs[0], rsems[0])
    pltpu.async_copy(src_hbm.at[pl.ds(ebase + _CH, _CH)], srcs[1], rsems[1])
    pltpu.async_copy(dst_hbm.at[pl.ds(ebase, _CH)], dsts[0], dsems[0])
    pltpu.async_copy(dst_hbm.at[pl.ds(ebase + _CH, _CH)], dsts[1], dsems[1])
    pltpu.async_copy(wts_hbm.at[base], wtss[0], wsems[0])
    pltpu.make_async_copy(src_hbm.at[pl.ds(ebase, _CH)], srcs[0],
                          rsems[0]).wait()
    pltpu.async_copy(z_hbm.at[srcs[0]], zrows[0], gsems[0])

    def chunk_step(k, j):
        if True:
            b2 = j % 2
            b4 = j % 4
            nx2 = (j + 1) % 2
            nx4 = (j + 1) % 4
            pf4 = (j + 2) % 4

            @pl.when(k >= 2)
            def _():
                pltpu.make_async_copy(msgs[b2], acc_sh.at[dsts[pf4]],
                                      ssems[b2]).wait()

            @pl.when(k + 2 < _NCHT)
            def _():
                off = ebase + (k + 2) * _CH
                pltpu.async_copy(src_hbm.at[pl.ds(off, _CH)], srcs[pf4],
                                 rsems[pf4])
                pltpu.async_copy(dst_hbm.at[pl.ds(off, _CH)], dsts[pf4],
                                 dsems[pf4])

            @pl.when(k + 1 < _NCHT)
            def _():
                pltpu.make_async_copy(
                    src_hbm.at[pl.ds(ebase + (k + 1) * _CH, _CH)], srcs[nx4],
                    rsems[nx4]).wait()
                pltpu.async_copy(z_hbm.at[srcs[nx4]], zrows[nx2], gsems[nx2])
                pltpu.async_copy(wts_hbm.at[base + k + 1], wtss[nx2],
                                 wsems[nx2])

            pltpu.make_async_copy(z_hbm.at[srcs[b4]], zrows[b2],
                                  gsems[b2]).wait()
            pltpu.make_async_copy(wts_hbm.at[base + k], wtss[b2],
                                  wsems[b2]).wait()
            pltpu.make_async_copy(dst_hbm.at[pl.ds(ebase + k * _CH, _CH)],
                                  dsts[b4], dsems[b4]).wait()

            zr = zrows[b2]
            mg = msgs[b2]
            wt = wtss[b2]

            def edge_body(e, carry2):
                rw = e // 2
                cw = (e % 2) * (K * _L)
                w0 = wt[rw, pl.ds(cw, _L)]
                w1 = wt[rw, pl.ds(cw + _L, _L)]
                w2 = wt[rw, pl.ds(cw + 2 * _L, _L)]
                w3 = wt[rw, pl.ds(cw + 3 * _L, _L)]
                for t in range(D_OUT // (2 * _L)):
                    def halves(i):
                        pr = plsc.bitcast(
                            zr[e, pl.ds(i * (D_OUT // 2) + _L * t, _L)],
                            jnp.bfloat16)
                        return plsc.unpack(
                            pr, format=plsc.PackFormat.INTERLEAVED)

                    a0, b0 = halves(0)
                    a1, b1 = halves(1)
                    a2, b2 = halves(2)
                    a3, b3 = halves(3)
                    va = w0 * a0 + w1 * a1 + w2 * a2 + w3 * a3
                    vb = w0 * b0 + w1 * b1 + w2 * b2 + w3 * b3
                    mg[e, pl.ds(2 * _L * t, _L)] = va
                    mg[e, pl.ds(2 * _L * t + _L, _L)] = vb
                return carry2

            lax.fori_loop(0, _CH, edge_body, 0, unroll=2)

            pltpu.async_copy(msgs[b2], acc_sh.at[dsts[b4]],
                             ssems[b2], add=True)

    def super_body(k2, carry):
        for j in range(4):
            chunk_step(k2 * 4 + j, j)
        return carry

    ntail = _NCHT % 4
    lax.fori_loop(0, _NCHT // 4, super_body, 0)
    for j in range(ntail):
        chunk_step(jnp.int32((_NCHT // 4) * 4 + j), j)
    pltpu.make_async_copy(msgs[_NCHT % 2], acc_sh.at[dsts[(_NCHT - 2) % 4]],
                          ssems[_NCHT % 2]).wait()
    pltpu.make_async_copy(msgs[(_NCHT + 1) % 2],
                          acc_sh.at[dsts[(_NCHT - 1) % 4]],
                          ssems[(_NCHT + 1) % 2]).wait()
    plsc.subcore_barrier()
    pltpu.sync_copy(acc_sh.at[pl.ds(s * _ROWS_PT, _ROWS_PT)],
                    out_hbm.at[c, pl.ds(s * _ROWS_PT, _ROWS_PT)])


def _sc_scatter(z, srcp, dstp, wts, zeros):
    mesh = plsc.VectorSubcoreMesh(core_axis_name="c", subcore_axis_name="s")
    fn = functools.partial(
        pl.kernel,
        mesh=mesh,
        compiler_params=pltpu.CompilerParams(needs_layout_passes=False),
        out_type=jax.ShapeDtypeStruct((_NC, _N_PAD, D_OUT), jnp.float32),
        scratch_types=(
            [pltpu.VMEM((_CH,), jnp.int32)] * 8
            + [pltpu.VMEM((_CH // 2, 2 * K * _L), jnp.float32)] * 2
            + [pltpu.VMEM((_CH, K * D_OUT // 2), jnp.int32)] * 2
            + [pltpu.VMEM((_CH, D_OUT), jnp.float32)] * 2
            + [pltpu.VMEM_SHARED((_N_PAD, D_OUT), jnp.float32)]
            + [pltpu.SemaphoreType.DMA] * 14
        ),
    )(_sc_body)
    return fn(z, srcp, dstp, wts, zeros)


def _final_body(p_ref, b_ref, out_ref):
    out_ref[...] = jnp.maximum(p_ref[0] + p_ref[1] + b_ref[...], 0.0)


def _final(partials, bias2d):
    grid = N_NODES // _N_BLK
    return pl.pallas_call(
        _final_body,
        grid=(grid,),
        in_specs=[
            pl.BlockSpec((_NC, _N_BLK, D_OUT), lambda i: (0, i, 0)),
            pl.BlockSpec((1, D_OUT), lambda i: (0, 0)),
        ],
        out_specs=pl.BlockSpec((_N_BLK, D_OUT), lambda i: (i, 0)),
        out_shape=jax.ShapeDtypeStruct((N_NODES, D_OUT), jnp.float32),
    )(partials, bias2d)


def kernel(x, edge_index, edge_attr, W1, W2, W3, W4, Wc, bias):
    src = edge_index[0].astype(jnp.int32)
    dst = edge_index[1].astype(jnp.int32)
    w123 = jnp.concatenate([W1.T, W2.T, W3.T], axis=1)
    ws1 = jnp.kron(jnp.eye(2, dtype=jnp.float32), w123)
    w4x = jnp.repeat(W4.T, _L, axis=1)
    ws2 = jnp.kron(jnp.eye(2, dtype=jnp.float32), w4x)

    wflat = jnp.transpose(Wc, (1, 0, 2)).reshape(D_IN, K * D_OUT)
    pos = jnp.arange(K * D_OUT // 2)
    blk, l = pos // _L, pos % _L
    perm = jnp.concatenate([blk * 2 * _L + l, blk * 2 * _L + _L + l])
    wflat = wflat[:, perm]

    ea2 = _edge_mlp(edge_attr.reshape(_E2, 32), ws1, ws2)
    z = _z_matmul(x, wflat)

    wts = ea2.reshape(_NCH_G, _CH // 2, 2 * K * _L)

    zeros = jnp.zeros((_N_PAD, D_OUT), jnp.float32)
    partials = _sc_scatter(z, src, dst, wts, zeros)
    return _final(partials, bias.reshape(1, D_OUT))
